# Initial kernel scaffold; baseline (speedup 1.0000x reference)
import jax
import jax.numpy as jnp
from jax.experimental import pallas as pl


def kernel(coordinates, active, occupancies, radial_densities):
    raise NotImplementedError("write your pallas kernel here")


# speedup vs baseline: 296.2813x; 296.2813x over previous
import dataclasses

import jax
import jax.numpy as jnp
from jax import lax
from jax.experimental import pallas as pl
from jax.experimental.pallas import tpu as pltpu
from jax.experimental.pallas import tpu_sc as plsc

RMAX = 3.0
RSTEP = 0.01
GRID = 128
WIN = 13
NRAD = 302
NRADP = 304
NATOMS = 512
NBATCH = 4
CHUNK = 4
NCHUNK = GRID // CHUNK
NTILES = 32
UNITS_PER_TILE = NBATCH * NCHUNK // NTILES
CHUNK_ELEMS = CHUNK * GRID * GRID
GRIDF = GRID * GRID * GRID


def _dilate_body(atomdat_hbm, rad_hbm, out_hbm,
                 tab_sh, chunk_v, ca_v, cb_v, cc_v, occ_v, tabA_v, cc_sm):
    sid = lax.axis_index("s")
    cid = lax.axis_index("c")
    wid = sid * 2 + cid
    b = wid // 8
    qbase = (wid % 8) * UNITS_PER_TILE

    @pl.when(sid < NBATCH)
    def _():
        pltpu.sync_copy(rad_hbm.at[sid], tab_sh.at[sid])

    plsc.subcore_barrier()

    pltpu.sync_copy(atomdat_hbm.at[b, 0], ca_v)
    pltpu.sync_copy(atomdat_hbm.at[b, 1], cb_v)
    pltpu.sync_copy(atomdat_hbm.at[b, 2], cc_v)
    pltpu.sync_copy(atomdat_hbm.at[b, 3], occ_v)

    @pl.loop(0, NATOMS // 16)
    def _(g):
        vec = cc_v[pl.ds(pl.multiple_of(g * 16, 16), 16)]
        for l in range(16):
            cc_sm[g * 16 + l] = vec[l]

    zeros16 = jnp.zeros((16,), jnp.float32)
    iota_i = lax.iota(jnp.int32, 16)
    iota_f = iota_i.astype(jnp.float32)

    def fulli(x):
        return jnp.full((16,), x, jnp.int32)

    def fullf(x):
        return jnp.full((16,), x, jnp.float32)

    def iceil_v(x):
        t = x.astype(jnp.int32)
        return jnp.where(x > t.astype(jnp.float32), t + 1, t)

    for uu in range(UNITS_PER_TILE):
        c_slab0 = (qbase + uu) * CHUNK

        @pl.loop(0, CHUNK_ELEMS // 16)
        def _(i):
            chunk_v[pl.ds(i * 16, 16)] = zeros16

        def atom_body(atom, carry):
            cc = cc_sm[atom]
            t = (cc - 6.0).astype(jnp.int32)
            c0 = jnp.where(cc - 6.0 > t.astype(jnp.float32), t + 1, t)
            ncm1 = (cc + 6.0).astype(jnp.int32) - c0
            dmv = (c_slab0 - c0) & 127
            tp = jnp.where(dmv >= 125, dmv - 128, dmv)
            jlo = jnp.maximum(tp, 0)
            jhi = jnp.minimum(jnp.minimum(tp + 3, WIN - 1), ncm1)
            go = ((dmv <= WIN - 1) | (dmv >= 125)) & (jhi >= jlo)

            @pl.when(go)
            def _():
                pltpu.sync_copy(rad_hbm.at[b, atom], tabA_v)
                fa = fulli(atom)
                cav = plsc.load_gather(ca_v, [fa])
                cbv = plsc.load_gather(cb_v, [fa])
                occv = plsc.load_gather(occ_v, [fa])
                a0v = iceil_v(cav - 6.0)
                nam1v = (cav + 6.0).astype(jnp.int32) - a0v
                das = (cav - a0v.astype(jnp.float32)) - iota_f
                dxa = 0.5 * das
                da2 = dxa * dxa
                ai_vec = (a0v + iota_i) & 127
                mka = iota_i <= nam1v
                b0v = iceil_v(cbv - 6.0)
                nbm1v = (cbv + 6.0).astype(jnp.int32) - b0v
                cb_off = cbv - b0v.astype(jnp.float32)

                for s in range(CHUNK):
                    j = (c_slab0 + s - c0) & 127
                    pv = (j <= WIN - 1) & (j <= ncm1)

                    @pl.when(pv)
                    def _(j=j, s=s):
                        dc = cc - (c0 + j).astype(jnp.float32)
                        hc = dc * dc
                        sbase_vec = fulli(s * (GRID * GRID)) + ai_vec

                        def row_body(i, rc):
                            i_f = i.astype(jnp.float32)
                            dbv = cb_off - fullf(i_f)
                            hv = (fullf(hc) + dbv * dbv) * 0.25
                            d2 = da2 + hv
                            m = (mka & (fulli(i) <= nbm1v)
                                 & (d2 <= RMAX * RMAX))
                            dmx = jnp.maximum(d2, 1e-30)
                            ib = lax.bitcast_convert_type(dmx, jnp.int32)
                            ib = jnp.int32(0x5F3759DF) - (ib >> 1)
                            y = lax.bitcast_convert_type(ib, jnp.float32)
                            hh = 0.5 * dmx
                            y = y * (1.5 - hh * y * y)
                            y = y * (1.5 - hh * y * y)
                            y = y * (1.5 - hh * y * y)
                            r = d2 * y
                            rad = r / RSTEP
                            low = jnp.minimum(rad.astype(jnp.int32),
                                              NRAD - 1)
                            high = jnp.minimum(low + 1, NRAD - 1)
                            wh = rad - low.astype(jnp.float32)
                            wl = 1.0 - wh
                            dl = plsc.load_gather(tabA_v, [low])
                            dh = plsc.load_gather(tabA_v, [high])
                            val = occv * (wl * dl + wh * dh)
                            biv = (b0v + fulli(i)) & 127
                            idx = sbase_vec + biv * GRID
                            plsc.addupdate_scatter(chunk_v, [idx], val,
                                                   mask=m)
                            return rc

                        lax.fori_loop(0, WIN, row_body, 0)

            return carry

        lax.fori_loop(0, NATOMS, atom_body, 0)

        pltpu.sync_copy(
            chunk_v,
            out_hbm.at[b, pl.ds(c_slab0 * GRID * GRID, CHUNK_ELEMS)])


@jax.jit
def kernel(coordinates, active, occupancies, radial_densities):
    occ = occupancies * active.astype(jnp.float32)
    atomdat = jnp.concatenate(
        [coordinates.transpose(0, 2, 1), occ[:, None, :]], axis=1)
    rad = jnp.pad(radial_densities, ((0, 0), (0, 0), (0, NRADP - NRAD)))
    mesh = plsc.VectorSubcoreMesh(core_axis_name="c", subcore_axis_name="s")
    cp = pltpu.CompilerParams()
    if "needs_layout_passes" in pltpu.CompilerParams.__dataclass_fields__:
        cp = dataclasses.replace(cp, needs_layout_passes=False)
    f = pl.kernel(
        _dilate_body,
        compiler_params=cp,
        out_type=jax.ShapeDtypeStruct((NBATCH, GRIDF), jnp.float32),
        mesh=mesh,
        scratch_types=[
            pltpu.VMEM_SHARED((NBATCH, NATOMS, NRADP), jnp.float32),
            pltpu.VMEM((CHUNK_ELEMS,), jnp.float32),
            pltpu.VMEM((NATOMS,), jnp.float32),
            pltpu.VMEM((NATOMS,), jnp.float32),
            pltpu.VMEM((NATOMS,), jnp.float32),
            pltpu.VMEM((NATOMS,), jnp.float32),
            pltpu.VMEM((NRADP,), jnp.float32),
            pltpu.SMEM((NATOMS,), jnp.float32),
        ],
    )
    return f(atomdat, rad).reshape(NBATCH, GRID, GRID, GRID)


# speedup vs baseline: 396.6073x; 1.3386x over previous
import dataclasses

import jax
import jax.numpy as jnp
from jax import lax
from jax.experimental import pallas as pl
from jax.experimental.pallas import tpu as pltpu
from jax.experimental.pallas import tpu_sc as plsc

RMAX = 3.0
RSTEP = 0.01
GRID = 128
WIN = 13
NRAD = 302
NRADP = 304
NATOMS = 512
NBATCH = 4
CHUNK = 4
NCHUNK = GRID // CHUNK
NTILES = 32
UNITS_PER_TILE = NBATCH * NCHUNK // NTILES
CHUNK_ELEMS = CHUNK * GRID * GRID
GRIDF = GRID * GRID * GRID


def _dilate_body(atomdat_hbm, rad_hbm, out_hbm,
                 chunk_v, ca_v, cb_v, cc_v, occ_v, tab0_v, tab1_v,
                 cc_sm, lst_sm, sem0, sem1):
    sid = lax.axis_index("s")
    cid = lax.axis_index("c")
    wid = sid * 2 + cid
    b = wid // 8
    qbase = (wid % 8) * UNITS_PER_TILE

    pltpu.sync_copy(atomdat_hbm.at[b, 0], ca_v)
    pltpu.sync_copy(atomdat_hbm.at[b, 1], cb_v)
    pltpu.sync_copy(atomdat_hbm.at[b, 2], cc_v)
    pltpu.sync_copy(atomdat_hbm.at[b, 3], occ_v)

    @pl.loop(0, NATOMS // 16)
    def _(g):
        vec = cc_v[pl.ds(pl.multiple_of(g * 16, 16), 16)]
        for l in range(16):
            cc_sm[g * 16 + l] = vec[l]

    zeros16 = jnp.zeros((16,), jnp.float32)
    iota_i = lax.iota(jnp.int32, 16)
    iota_f = iota_i.astype(jnp.float32)

    def fulli(x):
        return jnp.full((16,), x, jnp.int32)

    def fullf(x):
        return jnp.full((16,), x, jnp.float32)

    def iceil_v(x):
        t = x.astype(jnp.int32)
        return jnp.where(x > t.astype(jnp.float32), t + 1, t)

    def c0_of(cc):
        t = (cc - 6.0).astype(jnp.int32)
        return jnp.where(cc - 6.0 > t.astype(jnp.float32), t + 1, t)

    for uu in range(UNITS_PER_TILE):
        c_slab0 = (qbase + uu) * CHUNK

        @pl.loop(0, CHUNK_ELEMS // 16)
        def _(i):
            chunk_v[pl.ds(i * 16, 16)] = zeros16

        def scan_body(atom, cnt):
            cc = cc_sm[atom]
            c0 = c0_of(cc)
            ncm1 = (cc + 6.0).astype(jnp.int32) - c0
            dmv = (c_slab0 - c0) & 127
            tp = jnp.where(dmv >= 125, dmv - 128, dmv)
            jlo = jnp.maximum(tp, 0)
            jhi = jnp.minimum(jnp.minimum(tp + 3, WIN - 1), ncm1)
            go = ((dmv <= WIN - 1) | (dmv >= 125)) & (jhi >= jlo)

            @pl.when(go)
            def _():
                lst_sm[cnt] = atom

            return cnt + jnp.where(go, 1, 0)

        cnt = lax.fori_loop(0, NATOMS, scan_body, 0)

        def process(atom, tab_v):
            cc = cc_sm[atom]
            c0 = c0_of(cc)
            ncm1 = (cc + 6.0).astype(jnp.int32) - c0
            fa = fulli(atom)
            cav = plsc.load_gather(ca_v, [fa])
            cbv = plsc.load_gather(cb_v, [fa])
            occv = plsc.load_gather(occ_v, [fa])
            a0v = iceil_v(cav - 6.0)
            nam1v = (cav + 6.0).astype(jnp.int32) - a0v
            das = (cav - a0v.astype(jnp.float32)) - iota_f
            dxa = 0.5 * das
            da2 = dxa * dxa
            ai_vec = (a0v + iota_i) & 127
            mka = iota_i <= nam1v
            b0v = iceil_v(cbv - 6.0)
            nbm1v = (cbv + 6.0).astype(jnp.int32) - b0v
            cb_off = cbv - b0v.astype(jnp.float32)

            for s in range(CHUNK):
                j = (c_slab0 + s - c0) & 127
                pv = (j <= WIN - 1) & (j <= ncm1)

                @pl.when(pv)
                def _(j=j, s=s):
                    dc = cc - (c0 + j).astype(jnp.float32)
                    hc = dc * dc
                    sbase_vec = fulli(s * (GRID * GRID)) + ai_vec

                    def row_body(i, rc):
                        i_f = i.astype(jnp.float32)
                        dbv = cb_off - fullf(i_f)
                        hv = (fullf(hc) + dbv * dbv) * 0.25
                        d2 = da2 + hv
                        m = (mka & (fulli(i) <= nbm1v)
                             & (d2 <= RMAX * RMAX))
                        dmx = jnp.maximum(d2, 1e-30)
                        ib = lax.bitcast_convert_type(dmx, jnp.int32)
                        ib = jnp.int32(0x5F3759DF) - (ib >> 1)
                        y = lax.bitcast_convert_type(ib, jnp.float32)
                        hh = 0.5 * dmx
                        y = y * (1.5 - hh * y * y)
                        y = y * (1.5 - hh * y * y)
                        y = y * (1.5 - hh * y * y)
                        r = d2 * y
                        rad = r * (1.0 / RSTEP)
                        low = jnp.minimum(rad.astype(jnp.int32), NRAD - 1)
                        high = jnp.minimum(low + 1, NRAD - 1)
                        wh = rad - low.astype(jnp.float32)
                        wl = 1.0 - wh
                        dl = plsc.load_gather(tab_v, [low])
                        dh = plsc.load_gather(tab_v, [high])
                        val = occv * (wl * dl + wh * dh)
                        biv = (b0v + fulli(i)) & 127
                        idx = sbase_vec + biv * GRID
                        plsc.addupdate_scatter(chunk_v, [idx], val, mask=m)
                        return rc

                    lax.fori_loop(0, WIN, row_body, 0, unroll=2)

        def start_fetch(k, tab_v, sem):
            pltpu.async_copy(rad_hbm.at[b, lst_sm[k]], tab_v, sem)

        def wait_fetch(tab_v, sem):
            pltpu.make_async_copy(rad_hbm.at[b, 0], tab_v, sem).wait()

        @pl.when(cnt > 0)
        def _():
            start_fetch(0, tab0_v, sem0)

        def pair_body(p, _):
            k = 2 * p

            @pl.when(k + 1 < cnt)
            def _():
                start_fetch(k + 1, tab1_v, sem1)

            wait_fetch(tab0_v, sem0)
            process(lst_sm[k], tab0_v)

            @pl.when(k + 2 < cnt)
            def _():
                start_fetch(k + 2, tab0_v, sem0)

            @pl.when(k + 1 < cnt)
            def _():
                wait_fetch(tab1_v, sem1)
                process(lst_sm[k + 1], tab1_v)

            return 0

        lax.fori_loop(0, (cnt + 1) // 2, pair_body, 0)

        pltpu.sync_copy(
            chunk_v,
            out_hbm.at[b, pl.ds(c_slab0 * GRID * GRID, CHUNK_ELEMS)])


@jax.jit
def kernel(coordinates, active, occupancies, radial_densities):
    occ = occupancies * active.astype(jnp.float32)
    atomdat = jnp.concatenate(
        [coordinates.transpose(0, 2, 1), occ[:, None, :]], axis=1)
    rad = jnp.pad(radial_densities, ((0, 0), (0, 0), (0, NRADP - NRAD)))
    mesh = plsc.VectorSubcoreMesh(core_axis_name="c", subcore_axis_name="s")
    cp = pltpu.CompilerParams()
    if "needs_layout_passes" in pltpu.CompilerParams.__dataclass_fields__:
        cp = dataclasses.replace(cp, needs_layout_passes=False)
    f = pl.kernel(
        _dilate_body,
        compiler_params=cp,
        out_type=jax.ShapeDtypeStruct((NBATCH, GRIDF), jnp.float32),
        mesh=mesh,
        scratch_types=[
            pltpu.VMEM((CHUNK_ELEMS,), jnp.float32),
            pltpu.VMEM((NATOMS,), jnp.float32),
            pltpu.VMEM((NATOMS,), jnp.float32),
            pltpu.VMEM((NATOMS,), jnp.float32),
            pltpu.VMEM((NATOMS,), jnp.float32),
            pltpu.VMEM((NRADP,), jnp.float32),
            pltpu.VMEM((NRADP,), jnp.float32),
            pltpu.SMEM((NATOMS,), jnp.float32),
            pltpu.SMEM((NATOMS,), jnp.int32),
            pltpu.SemaphoreType.DMA,
            pltpu.SemaphoreType.DMA,
        ],
    )
    return f(atomdat, rad).reshape(NBATCH, GRID, GRID, GRID)
